# Initial kernel scaffold; baseline (speedup 1.0000x reference)
import jax
import jax.numpy as jnp
from jax.experimental import pallas as pl


def kernel(conv1_w, conv1_b, w1_0_0, b1_0_0, w2_0_0, b2_0_0, w1_0_1, b1_0_1, w2_0_1, b2_0_1, w1_1_0, b1_1_0, mask1_1_0, w2_1_0, b2_1_0, w1_1_1, b1_1_1, w2_1_1, b2_1_1, w1_2_0, b1_2_0, mask1_2_0, w2_2_0, b2_2_0, w1_2_1, b1_2_1, w2_2_1, b2_2_1, w1_3_0, b1_3_0, mask1_3_0, w2_3_0, b2_3_0, w1_3_1, b1_3_1, w2_3_1, b2_3_1, fc_w, fc_b, x, labels):
    raise NotImplementedError("write your pallas kernel here")


# speedup vs baseline: 2.6245x; 2.6245x over previous
import jax
import jax.numpy as jnp
from jax import lax
from jax.experimental import pallas as pl
from jax.experimental.pallas import tpu as pltpu

_BF16 = jnp.bfloat16
_MB = 1024 * 1024


def _pad_hw(x, val=0.0):
    TB, H, W, C = x.shape
    dt = x.dtype
    zc = jnp.full((TB, H, 1, C), val, dt)
    x = jnp.concatenate([zc, x, zc], axis=2)
    zr = jnp.full((TB, 1, W + 2, C), val, dt)
    return jnp.concatenate([zr, x, zr], axis=1)


def _conv3x3(xp, w_ref, OH, OW):
    TB = xp.shape[0]
    C = xp.shape[3]
    N = w_ref.shape[1]
    M = TB * OH * OW
    acc = jnp.zeros((M, N), jnp.float32)
    for t in range(9):
        i, j = t // 3, t % 3
        xs = xp[:, i:i + OH, j:j + OW, :].reshape(M, C)
        acc = acc + jnp.dot(xs, w_ref[t * C:(t + 1) * C, :],
                            preferred_element_type=jnp.float32)
    return acc


def _phase_split(x):
    ee = x[:, 0::2, 0::2, :]
    eo = jnp.pad(x[:, 0::2, 1::2, :], ((0, 0), (0, 0), (1, 0), (0, 0)))
    oe = jnp.pad(x[:, 1::2, 0::2, :], ((0, 0), (1, 0), (0, 0), (0, 0)))
    oo = jnp.pad(x[:, 1::2, 1::2, :], ((0, 0), (1, 0), (1, 0), (0, 0)))
    return ee, eo, oe, oo


def _conv3x3_s2(pee, peo, poe, poo, w_ref, OH, OW):
    TB = pee.shape[0]
    C = pee.shape[3]
    N = w_ref.shape[1]
    M = TB * OH * OW
    acc = jnp.zeros((M, N), jnp.float32)
    for t in range(9):
        di, dj = t // 3, t % 3
        ro, co = di // 2, dj // 2
        if di == 1 and dj == 1:
            v = pee
        elif di == 1:
            v = peo[:, :, co:co + OW, :]
        elif dj == 1:
            v = poe[:, ro:ro + OH, :, :]
        else:
            v = poo[:, ro:ro + OH, co:co + OW, :]
        acc = acc + jnp.dot(v.reshape(M, C), w_ref[t * C:(t + 1) * C, :],
                            preferred_element_type=jnp.float32)
    return acc


def _block_a_kernel(x_ref, w1_ref, b1_ref, w2_ref, b2_ref, o_ref):
    TB, H, W, C = x_ref.shape
    x = x_ref[...]
    h = _conv3x3(_pad_hw(x), w1_ref, H, W) + b1_ref[...]
    h = jnp.maximum(h, 0.0).astype(_BF16).reshape(TB, H, W, C)
    acc2 = _conv3x3(_pad_hw(h), w2_ref, H, W) + b2_ref[...]
    out = jnp.maximum(acc2 + x.reshape(TB * H * W, C).astype(jnp.float32), 0.0)
    o_ref[...] = out.astype(_BF16).reshape(TB, H, W, C)


def _block_b_kernel(ee_ref, eo_ref, oe_ref, oo_ref, w1_ref, b1_ref, w2_ref,
                    b2_ref, o_ref, *, cout):
    TB, OH, OW, _ = ee_ref.shape
    out1 = _conv3x3_s2(ee_ref[...], eo_ref[...], oe_ref[...], oo_ref[...],
                       w1_ref, OH, OW) + b1_ref[...]
    h = jnp.maximum(out1[:, :cout], 0.0).astype(_BF16)
    idn = out1[:, cout:].astype(_BF16)
    hp = _pad_hw(h.reshape(TB, OH, OW, cout))
    acc2 = _conv3x3(hp, w2_ref, OH, OW) + b2_ref[...]
    out = jnp.maximum(acc2 + idn.astype(jnp.float32), 0.0)
    o_ref[...] = out.astype(_BF16).reshape(TB, OH, OW, cout)


def _run_block_a(x, w1, b1, w2, b2, tb, vmem_mb):
    B, H, W, C = x.shape
    return pl.pallas_call(
        _block_a_kernel,
        out_shape=jax.ShapeDtypeStruct((B, H, W, C), _BF16),
        grid=(B // tb,),
        in_specs=[pl.BlockSpec((tb, H, W, C), lambda i: (i, 0, 0, 0)),
                  pl.BlockSpec(w1.shape, lambda i: (0, 0)),
                  pl.BlockSpec(b1.shape, lambda i: (0, 0)),
                  pl.BlockSpec(w2.shape, lambda i: (0, 0)),
                  pl.BlockSpec(b2.shape, lambda i: (0, 0))],
        out_specs=pl.BlockSpec((tb, H, W, C), lambda i: (i, 0, 0, 0)),
        compiler_params=pltpu.CompilerParams(
            dimension_semantics=("parallel",),
            vmem_limit_bytes=vmem_mb * _MB),
    )(x, w1, b1, w2, b2)


def _run_block_b(x, w1, b1, w2, b2, cout, tb, vmem_mb):
    import functools
    B, H, W, Cin = x.shape
    OH, OW = H // 2, W // 2
    phases = _phase_split(x)
    in_specs = [pl.BlockSpec((tb,) + p.shape[1:], lambda i: (i, 0, 0, 0))
                for p in phases]
    in_specs += [pl.BlockSpec(w1.shape, lambda i: (0, 0)),
                 pl.BlockSpec(b1.shape, lambda i: (0, 0)),
                 pl.BlockSpec(w2.shape, lambda i: (0, 0)),
                 pl.BlockSpec(b2.shape, lambda i: (0, 0))]
    return pl.pallas_call(
        functools.partial(_block_b_kernel, cout=cout),
        out_shape=jax.ShapeDtypeStruct((B, OH, OW, cout), _BF16),
        grid=(B // tb,),
        in_specs=in_specs,
        out_specs=pl.BlockSpec((tb, OH, OW, cout), lambda i: (i, 0, 0, 0)),
        compiler_params=pltpu.CompilerParams(
            dimension_semantics=("parallel",),
            vmem_limit_bytes=vmem_mb * _MB),
    )(*phases, w1, b1, w2, b2)


def _conv1_pool_kernel(p_ref, w_ref, b_ref, o_ref):
    acc = jnp.dot(p_ref[0], w_ref[...], preferred_element_type=jnp.float32)
    out = jnp.maximum(acc + b_ref[...], 0.0).astype(_BF16)
    img = out.reshape(112, 112, 128)
    neg = float(jnp.finfo(_BF16).min)
    zc = jnp.full((112, 1, 128), neg, _BF16)
    img = jnp.concatenate([zc, img, zc], axis=1)
    zr = jnp.full((1, 114, 128), neg, _BF16)
    img = jnp.concatenate([zr, img, zr], axis=0)
    m = None
    for i in range(3):
        for j in range(3):
            t = img[i:i + 112, j:j + 112, :]
            t = t.reshape(56, 2, 112, 128)[:, 0]
            t = t.reshape(56, 56, 2, 128)[:, :, 0]
            m = t if m is None else jnp.maximum(m, t)
    o_ref[0] = m


def _im2col7(x):
    B = x.shape[0]
    x = jnp.pad(x, ((0, 0), (3, 3), (3, 3), (0, 0)))
    cols = []
    for i in range(7):
        for j in range(7):
            cols.append(lax.slice(
                x, (0, i, j, 0), (B, i + 223, j + 223, 3), (1, 2, 2, 1)))
    patches = jnp.stack(cols, axis=3).reshape(B, 112 * 112, 147)
    return jnp.pad(patches, ((0, 0), (0, 0), (0, 109)))


def _conv1_pool(x_nhwc, w, b):
    B = x_nhwc.shape[0]
    patches = _im2col7(x_nhwc)
    return pl.pallas_call(
        _conv1_pool_kernel,
        out_shape=jax.ShapeDtypeStruct((B, 56, 56, 128), _BF16),
        grid=(B,),
        in_specs=[pl.BlockSpec((1, 112 * 112, 256), lambda i: (i, 0, 0)),
                  pl.BlockSpec((256, 128), lambda i: (0, 0)),
                  pl.BlockSpec((1, 128), lambda i: (0, 0))],
        out_specs=pl.BlockSpec((1, 56, 56, 128), lambda i: (i, 0, 0, 0)),
        compiler_params=pltpu.CompilerParams(
            dimension_semantics=("parallel",),
            vmem_limit_bytes=48 * _MB),
    )(patches, w, b)


def _head_kernel(x_ref, w_ref, b_ref, l_ref, loss_ref, pred_ref):
    pooled = jnp.mean(x_ref[...].astype(jnp.float32), axis=1)
    logits = jnp.dot(pooled.astype(_BF16), w_ref[...],
                     preferred_element_type=jnp.float32) + b_ref[...]
    B, Np = logits.shape
    col = lax.broadcasted_iota(jnp.int32, (B, Np), 1)
    lg = jnp.where(col < 1000, logits, -jnp.inf)
    row_max = jnp.max(lg, axis=-1, keepdims=True)
    lse = jnp.log(jnp.sum(jnp.exp(lg - row_max), axis=-1,
                          keepdims=True)) + row_max
    onehot = (col == l_ref[...]).astype(jnp.float32)
    picked = jnp.sum(logits * onehot, axis=-1, keepdims=True)
    loss_ref[...] = jnp.sum(lse - picked, axis=0, keepdims=True) / B
    masked = jnp.where(lg == row_max, col, Np)
    pred_ref[...] = jnp.min(masked, axis=-1, keepdims=True)


def _head(x, fc_w, fc_b, labels):
    B, HW, C = x.shape
    Np = fc_w.shape[1]
    labels2 = labels.reshape(B, 1).astype(jnp.int32)
    loss, pred = pl.pallas_call(
        _head_kernel,
        out_shape=(jax.ShapeDtypeStruct((1, 1), jnp.float32),
                   jax.ShapeDtypeStruct((B, 1), jnp.int32)),
        grid=(1,),
        in_specs=[pl.BlockSpec((B, HW, C), lambda i: (0, 0, 0)),
                  pl.BlockSpec((C, Np), lambda i: (0, 0)),
                  pl.BlockSpec((1, Np), lambda i: (0, 0)),
                  pl.BlockSpec((B, 1), lambda i: (0, 0))],
        out_specs=(pl.BlockSpec((1, 1), lambda i: (0, 0)),
                   pl.BlockSpec((B, 1), lambda i: (0, 0))),
    )(x, fc_w, fc_b, labels2)
    return loss[0, 0], pred


def kernel(conv1_w, conv1_b,
           w1_0_0, b1_0_0, w2_0_0, b2_0_0,
           w1_0_1, b1_0_1, w2_0_1, b2_0_1,
           w1_1_0, b1_1_0, mask1_1_0, w2_1_0, b2_1_0,
           w1_1_1, b1_1_1, w2_1_1, b2_1_1,
           w1_2_0, b1_2_0, mask1_2_0, w2_2_0, b2_2_0,
           w1_2_1, b1_2_1, w2_2_1, b2_2_1,
           w1_3_0, b1_3_0, mask1_3_0, w2_3_0, b2_3_0,
           w1_3_1, b1_3_1, w2_3_1, b2_3_1,
           fc_w, fc_b, x, labels):
    del mask1_1_0, mask1_2_0, mask1_3_0
    x_nhwc = jnp.transpose(x, (0, 2, 3, 1)).astype(_BF16)

    h = _conv1_pool(x_nhwc, conv1_w, conv1_b)

    h = _run_block_a(h, w1_0_0, b1_0_0, w2_0_0, b2_0_0, tb=2, vmem_mb=48)
    h = _run_block_a(h, w1_0_1, b1_0_1, w2_0_1, b2_0_1, tb=2, vmem_mb=48)

    h = _run_block_b(h, w1_1_0, b1_1_0, w2_1_0, b2_1_0, cout=128, tb=4,
                     vmem_mb=48)
    h = _run_block_a(h, w1_1_1, b1_1_1, w2_1_1, b2_1_1, tb=8, vmem_mb=48)

    h = _run_block_b(h, w1_2_0, b1_2_0, w2_2_0, b2_2_0, cout=256, tb=8,
                     vmem_mb=48)
    h = _run_block_a(h, w1_2_1, b1_2_1, w2_2_1, b2_2_1, tb=8, vmem_mb=48)

    h = _run_block_b(h, w1_3_0, b1_3_0, w2_3_0, b2_3_0, cout=512, tb=16,
                     vmem_mb=48)
    h = _run_block_a(h, w1_3_1, b1_3_1, w2_3_1, b2_3_1, tb=16, vmem_mb=48)

    B = h.shape[0]
    loss, preds = _head(h.reshape(B, 49, 512), fc_w, fc_b, labels)
    out = {'y_pred': preds, 'y_true': labels.reshape(-1, 1)}
    return out, loss


# speedup vs baseline: 2.7758x; 1.0576x over previous
import jax
import jax.numpy as jnp
from jax import lax
from jax.experimental import pallas as pl
from jax.experimental.pallas import tpu as pltpu

_BF16 = jnp.bfloat16
_MB = 1024 * 1024


def _pad_hw(x, val=0.0):
    TB, H, W, C = x.shape
    dt = x.dtype
    zc = jnp.full((TB, H, 1, C), val, dt)
    x = jnp.concatenate([zc, x, zc], axis=2)
    zr = jnp.full((TB, 1, W + 2, C), val, dt)
    return jnp.concatenate([zr, x, zr], axis=1)


def _conv3x3(xp, w_ref, OH, OW):
    TB = xp.shape[0]
    C = xp.shape[3]
    M = TB * OH * OW
    xjs = [xp[:, :, j:j + OW, :] for j in range(3)]
    cols = []
    for t in range(9):
        i, j = t // 3, t % 3
        cols.append(xjs[j][:, i:i + OH, :, :].reshape(M, C))
    patches = jnp.concatenate(cols, axis=1)
    return jnp.dot(patches, w_ref[...], preferred_element_type=jnp.float32)


def _phase_split(x):
    ee = x[:, 0::2, 0::2, :]
    eo = jnp.pad(x[:, 0::2, 1::2, :], ((0, 0), (0, 0), (1, 0), (0, 0)))
    oe = jnp.pad(x[:, 1::2, 0::2, :], ((0, 0), (1, 0), (0, 0), (0, 0)))
    oo = jnp.pad(x[:, 1::2, 1::2, :], ((0, 0), (1, 0), (1, 0), (0, 0)))
    return ee, eo, oe, oo


def _conv3x3_s2(pee, peo, poe, poo, w_ref, OH, OW):
    TB = pee.shape[0]
    C = pee.shape[3]
    M = TB * OH * OW
    cols = []
    for t in range(9):
        di, dj = t // 3, t % 3
        ro, co = di // 2, dj // 2
        if di == 1 and dj == 1:
            v = pee
        elif di == 1:
            v = peo[:, :, co:co + OW, :]
        elif dj == 1:
            v = poe[:, ro:ro + OH, :, :]
        else:
            v = poo[:, ro:ro + OH, co:co + OW, :]
        cols.append(v.reshape(M, C))
    patches = jnp.concatenate(cols, axis=1)
    return jnp.dot(patches, w_ref[...], preferred_element_type=jnp.float32)


def _block_a_kernel(x_ref, w1_ref, b1_ref, w2_ref, b2_ref, o_ref):
    TB, H, W, C = x_ref.shape
    x = x_ref[...]
    h = _conv3x3(_pad_hw(x), w1_ref, H, W) + b1_ref[...]
    h = jnp.maximum(h, 0.0).astype(_BF16).reshape(TB, H, W, C)
    acc2 = _conv3x3(_pad_hw(h), w2_ref, H, W) + b2_ref[...]
    out = jnp.maximum(acc2 + x.reshape(TB * H * W, C).astype(jnp.float32), 0.0)
    o_ref[...] = out.astype(_BF16).reshape(TB, H, W, C)


def _block_b_kernel(ee_ref, eo_ref, oe_ref, oo_ref, w1_ref, b1_ref, w2_ref,
                    b2_ref, o_ref, *, cout):
    TB, OH, OW, _ = ee_ref.shape
    out1 = _conv3x3_s2(ee_ref[...], eo_ref[...], oe_ref[...], oo_ref[...],
                       w1_ref, OH, OW) + b1_ref[...]
    h = jnp.maximum(out1[:, :cout], 0.0).astype(_BF16)
    idn = out1[:, cout:].astype(_BF16)
    hp = _pad_hw(h.reshape(TB, OH, OW, cout))
    acc2 = _conv3x3(hp, w2_ref, OH, OW) + b2_ref[...]
    out = jnp.maximum(acc2 + idn.astype(jnp.float32), 0.0)
    o_ref[...] = out.astype(_BF16).reshape(TB, OH, OW, cout)


def _run_block_a(x, w1, b1, w2, b2, tb, vmem_mb):
    B, H, W, C = x.shape
    return pl.pallas_call(
        _block_a_kernel,
        out_shape=jax.ShapeDtypeStruct((B, H, W, C), _BF16),
        grid=(B // tb,),
        in_specs=[pl.BlockSpec((tb, H, W, C), lambda i: (i, 0, 0, 0)),
                  pl.BlockSpec(w1.shape, lambda i: (0, 0)),
                  pl.BlockSpec(b1.shape, lambda i: (0, 0)),
                  pl.BlockSpec(w2.shape, lambda i: (0, 0)),
                  pl.BlockSpec(b2.shape, lambda i: (0, 0))],
        out_specs=pl.BlockSpec((tb, H, W, C), lambda i: (i, 0, 0, 0)),
        compiler_params=pltpu.CompilerParams(
            dimension_semantics=("parallel",),
            vmem_limit_bytes=vmem_mb * _MB),
    )(x, w1, b1, w2, b2)


def _run_block_b(x, w1, b1, w2, b2, cout, tb, vmem_mb):
    import functools
    B, H, W, Cin = x.shape
    OH, OW = H // 2, W // 2
    phases = _phase_split(x)
    in_specs = [pl.BlockSpec((tb,) + p.shape[1:], lambda i: (i, 0, 0, 0))
                for p in phases]
    in_specs += [pl.BlockSpec(w1.shape, lambda i: (0, 0)),
                 pl.BlockSpec(b1.shape, lambda i: (0, 0)),
                 pl.BlockSpec(w2.shape, lambda i: (0, 0)),
                 pl.BlockSpec(b2.shape, lambda i: (0, 0))]
    return pl.pallas_call(
        functools.partial(_block_b_kernel, cout=cout),
        out_shape=jax.ShapeDtypeStruct((B, OH, OW, cout), _BF16),
        grid=(B // tb,),
        in_specs=in_specs,
        out_specs=pl.BlockSpec((tb, OH, OW, cout), lambda i: (i, 0, 0, 0)),
        compiler_params=pltpu.CompilerParams(
            dimension_semantics=("parallel",),
            vmem_limit_bytes=vmem_mb * _MB),
    )(*phases, w1, b1, w2, b2)


def _conv1_pool_kernel(p_ref, w_ref, b_ref, o_ref):
    acc = jnp.dot(p_ref[0], w_ref[...], preferred_element_type=jnp.float32)
    out = jnp.maximum(acc + b_ref[...], 0.0).astype(_BF16)
    img = out.reshape(112, 112, 128)
    neg = float(jnp.finfo(_BF16).min)
    zr = jnp.full((1, 112, 128), neg, _BF16)
    a = jnp.concatenate([zr, img, zr], axis=0)
    a = jnp.maximum(jnp.maximum(a[0:112], a[1:113]), a[2:114])
    a = a.reshape(56, 2, 112, 128)[:, 0]
    zc = jnp.full((56, 1, 128), neg, _BF16)
    b = jnp.concatenate([zc, a, zc], axis=1)
    b = jnp.maximum(jnp.maximum(b[:, 0:112], b[:, 1:113]), b[:, 2:114])
    o_ref[0] = b.reshape(56, 56, 2, 128)[:, :, 0]


def _im2col7(x):
    B = x.shape[0]
    x = jnp.pad(x, ((0, 0), (3, 3), (3, 3), (0, 0)))
    cols = []
    for i in range(7):
        for j in range(7):
            cols.append(lax.slice(
                x, (0, i, j, 0), (B, i + 223, j + 223, 3), (1, 2, 2, 1)))
    patches = jnp.stack(cols, axis=3).reshape(B, 112 * 112, 147)
    return jnp.pad(patches, ((0, 0), (0, 0), (0, 109)))


def _conv1_pool(x_nhwc, w, b):
    B = x_nhwc.shape[0]
    patches = _im2col7(x_nhwc)
    return pl.pallas_call(
        _conv1_pool_kernel,
        out_shape=jax.ShapeDtypeStruct((B, 56, 56, 128), _BF16),
        grid=(B,),
        in_specs=[pl.BlockSpec((1, 112 * 112, 256), lambda i: (i, 0, 0)),
                  pl.BlockSpec((256, 128), lambda i: (0, 0)),
                  pl.BlockSpec((1, 128), lambda i: (0, 0))],
        out_specs=pl.BlockSpec((1, 56, 56, 128), lambda i: (i, 0, 0, 0)),
        compiler_params=pltpu.CompilerParams(
            dimension_semantics=("parallel",),
            vmem_limit_bytes=48 * _MB),
    )(patches, w, b)


def _head_kernel(x_ref, w_ref, b_ref, l_ref, loss_ref, pred_ref):
    pooled = jnp.mean(x_ref[...].astype(jnp.float32), axis=1)
    logits = jnp.dot(pooled.astype(_BF16), w_ref[...],
                     preferred_element_type=jnp.float32) + b_ref[...]
    B, Np = logits.shape
    col = lax.broadcasted_iota(jnp.int32, (B, Np), 1)
    lg = jnp.where(col < 1000, logits, -jnp.inf)
    row_max = jnp.max(lg, axis=-1, keepdims=True)
    lse = jnp.log(jnp.sum(jnp.exp(lg - row_max), axis=-1,
                          keepdims=True)) + row_max
    onehot = (col == l_ref[...]).astype(jnp.float32)
    picked = jnp.sum(logits * onehot, axis=-1, keepdims=True)
    loss_ref[...] = jnp.sum(lse - picked, axis=0, keepdims=True) / B
    masked = jnp.where(lg == row_max, col, Np)
    pred_ref[...] = jnp.min(masked, axis=-1, keepdims=True)


def _head(x, fc_w, fc_b, labels):
    B, HW, C = x.shape
    Np = fc_w.shape[1]
    labels2 = labels.reshape(B, 1).astype(jnp.int32)
    loss, pred = pl.pallas_call(
        _head_kernel,
        out_shape=(jax.ShapeDtypeStruct((1, 1), jnp.float32),
                   jax.ShapeDtypeStruct((B, 1), jnp.int32)),
        grid=(1,),
        in_specs=[pl.BlockSpec((B, HW, C), lambda i: (0, 0, 0)),
                  pl.BlockSpec((C, Np), lambda i: (0, 0)),
                  pl.BlockSpec((1, Np), lambda i: (0, 0)),
                  pl.BlockSpec((B, 1), lambda i: (0, 0))],
        out_specs=(pl.BlockSpec((1, 1), lambda i: (0, 0)),
                   pl.BlockSpec((B, 1), lambda i: (0, 0))),
    )(x, fc_w, fc_b, labels2)
    return loss[0, 0], pred


def kernel(conv1_w, conv1_b,
           w1_0_0, b1_0_0, w2_0_0, b2_0_0,
           w1_0_1, b1_0_1, w2_0_1, b2_0_1,
           w1_1_0, b1_1_0, mask1_1_0, w2_1_0, b2_1_0,
           w1_1_1, b1_1_1, w2_1_1, b2_1_1,
           w1_2_0, b1_2_0, mask1_2_0, w2_2_0, b2_2_0,
           w1_2_1, b1_2_1, w2_2_1, b2_2_1,
           w1_3_0, b1_3_0, mask1_3_0, w2_3_0, b2_3_0,
           w1_3_1, b1_3_1, w2_3_1, b2_3_1,
           fc_w, fc_b, x, labels):
    del mask1_1_0, mask1_2_0, mask1_3_0
    x_nhwc = jnp.transpose(x.astype(_BF16), (0, 2, 3, 1))

    h = _conv1_pool(x_nhwc, conv1_w, conv1_b)

    h = _run_block_a(h, w1_0_0, b1_0_0, w2_0_0, b2_0_0, tb=2, vmem_mb=48)
    h = _run_block_a(h, w1_0_1, b1_0_1, w2_0_1, b2_0_1, tb=2, vmem_mb=48)

    h = _run_block_b(h, w1_1_0, b1_1_0, w2_1_0, b2_1_0, cout=128, tb=4,
                     vmem_mb=48)
    h = _run_block_a(h, w1_1_1, b1_1_1, w2_1_1, b2_1_1, tb=8, vmem_mb=48)

    h = _run_block_b(h, w1_2_0, b1_2_0, w2_2_0, b2_2_0, cout=256, tb=8,
                     vmem_mb=48)
    h = _run_block_a(h, w1_2_1, b1_2_1, w2_2_1, b2_2_1, tb=8, vmem_mb=48)

    h = _run_block_b(h, w1_3_0, b1_3_0, w2_3_0, b2_3_0, cout=512, tb=16,
                     vmem_mb=48)
    h = _run_block_a(h, w1_3_1, b1_3_1, w2_3_1, b2_3_1, tb=16, vmem_mb=48)

    B = h.shape[0]
    loss, preds = _head(h.reshape(B, 49, 512), fc_w, fc_b, labels)
    out = {'y_pred': preds, 'y_true': labels.reshape(-1, 1)}
    return out, loss


# speedup vs baseline: 3.0624x; 1.1033x over previous
import jax
import jax.numpy as jnp
from jax import lax
from jax.experimental import pallas as pl
from jax.experimental.pallas import tpu as pltpu

_BF16 = jnp.bfloat16
_MB = 1024 * 1024


def _pad_hw(x, val=0.0):
    TB, H, W, C = x.shape
    dt = x.dtype
    zc = jnp.full((TB, H, 1, C), val, dt)
    x = jnp.concatenate([zc, x, zc], axis=2)
    zr = jnp.full((TB, 1, W + 2, C), val, dt)
    return jnp.concatenate([zr, x, zr], axis=1)


def _conv3x3(xp, w_ref, OH, OW):
    TB = xp.shape[0]
    C = xp.shape[3]
    M = TB * OH * OW
    xjs = [xp[:, :, j:j + OW, :] for j in range(3)]
    cols = []
    for t in range(9):
        i, j = t // 3, t % 3
        cols.append(xjs[j][:, i:i + OH, :, :].reshape(M, C))
    patches = jnp.concatenate(cols, axis=1)
    return jnp.dot(patches, w_ref[...], preferred_element_type=jnp.float32)


def _phase_split(x):
    ee = x[:, 0::2, 0::2, :]
    eo = jnp.pad(x[:, 0::2, 1::2, :], ((0, 0), (0, 0), (1, 0), (0, 0)))
    oe = jnp.pad(x[:, 1::2, 0::2, :], ((0, 0), (1, 0), (0, 0), (0, 0)))
    oo = jnp.pad(x[:, 1::2, 1::2, :], ((0, 0), (1, 0), (1, 0), (0, 0)))
    return ee, eo, oe, oo


def _conv3x3_s2(pee, peo, poe, poo, w_ref, OH, OW):
    TB = pee.shape[0]
    C = pee.shape[3]
    M = TB * OH * OW
    cols = []
    for t in range(9):
        di, dj = t // 3, t % 3
        ro, co = di // 2, dj // 2
        if di == 1 and dj == 1:
            v = pee
        elif di == 1:
            v = peo[:, :, co:co + OW, :]
        elif dj == 1:
            v = poe[:, ro:ro + OH, :, :]
        else:
            v = poo[:, ro:ro + OH, co:co + OW, :]
        cols.append(v.reshape(M, C))
    patches = jnp.concatenate(cols, axis=1)
    return jnp.dot(patches, w_ref[...], preferred_element_type=jnp.float32)


def _block_a_kernel(x_ref, w1_ref, b1_ref, w2_ref, b2_ref, o_ref):
    TB, H, W, C = x_ref.shape
    x = x_ref[...]
    h = _conv3x3(_pad_hw(x), w1_ref, H, W) + b1_ref[...]
    h = jnp.maximum(h, 0.0).astype(_BF16).reshape(TB, H, W, C)
    acc2 = _conv3x3(_pad_hw(h), w2_ref, H, W) + b2_ref[...]
    out = jnp.maximum(acc2 + x.reshape(TB * H * W, C).astype(jnp.float32), 0.0)
    o_ref[...] = out.astype(_BF16).reshape(TB, H, W, C)


def _block_b_kernel(ee_ref, eo_ref, oe_ref, oo_ref, w1_ref, b1_ref, w2_ref,
                    b2_ref, o_ref, *, cout):
    TB, OH, OW, _ = ee_ref.shape
    out1 = _conv3x3_s2(ee_ref[...], eo_ref[...], oe_ref[...], oo_ref[...],
                       w1_ref, OH, OW) + b1_ref[...]
    h = jnp.maximum(out1[:, :cout], 0.0).astype(_BF16)
    idn = out1[:, cout:].astype(_BF16)
    hp = _pad_hw(h.reshape(TB, OH, OW, cout))
    acc2 = _conv3x3(hp, w2_ref, OH, OW) + b2_ref[...]
    out = jnp.maximum(acc2 + idn.astype(jnp.float32), 0.0)
    o_ref[...] = out.astype(_BF16).reshape(TB, OH, OW, cout)


def _run_block_a(x, w1, b1, w2, b2, tb, vmem_mb):
    B, H, W, C = x.shape
    return pl.pallas_call(
        _block_a_kernel,
        out_shape=jax.ShapeDtypeStruct((B, H, W, C), _BF16),
        grid=(B // tb,),
        in_specs=[pl.BlockSpec((tb, H, W, C), lambda i: (i, 0, 0, 0)),
                  pl.BlockSpec(w1.shape, lambda i: (0, 0)),
                  pl.BlockSpec(b1.shape, lambda i: (0, 0)),
                  pl.BlockSpec(w2.shape, lambda i: (0, 0)),
                  pl.BlockSpec(b2.shape, lambda i: (0, 0))],
        out_specs=pl.BlockSpec((tb, H, W, C), lambda i: (i, 0, 0, 0)),
        compiler_params=pltpu.CompilerParams(
            dimension_semantics=("parallel",),
            vmem_limit_bytes=vmem_mb * _MB),
    )(x, w1, b1, w2, b2)


def _run_block_b(x, w1, b1, w2, b2, cout, tb, vmem_mb):
    import functools
    B, H, W, Cin = x.shape
    OH, OW = H // 2, W // 2
    phases = _phase_split(x)
    in_specs = [pl.BlockSpec((tb,) + p.shape[1:], lambda i: (i, 0, 0, 0))
                for p in phases]
    in_specs += [pl.BlockSpec(w1.shape, lambda i: (0, 0)),
                 pl.BlockSpec(b1.shape, lambda i: (0, 0)),
                 pl.BlockSpec(w2.shape, lambda i: (0, 0)),
                 pl.BlockSpec(b2.shape, lambda i: (0, 0))]
    return pl.pallas_call(
        functools.partial(_block_b_kernel, cout=cout),
        out_shape=jax.ShapeDtypeStruct((B, OH, OW, cout), _BF16),
        grid=(B // tb,),
        in_specs=in_specs,
        out_specs=pl.BlockSpec((tb, OH, OW, cout), lambda i: (i, 0, 0, 0)),
        compiler_params=pltpu.CompilerParams(
            dimension_semantics=("parallel",),
            vmem_limit_bytes=vmem_mb * _MB),
    )(*phases, w1, b1, w2, b2)


def _conv1_pool_kernel(p_ref, w_ref, b_ref, o_ref):
    acc = jnp.dot(p_ref[0], w_ref[...], preferred_element_type=jnp.float32)
    out = jnp.maximum(acc + b_ref[...], 0.0).astype(_BF16)
    img = out.reshape(112, 112, 128)
    neg = float(jnp.finfo(_BF16).min)
    zr = jnp.full((1, 112, 128), neg, _BF16)
    a = jnp.concatenate([zr, img, zr], axis=0)
    a = jnp.maximum(jnp.maximum(a[0:112], a[1:113]), a[2:114])
    a = a.reshape(56, 2, 112, 128)[:, 0]
    zc = jnp.full((56, 1, 128), neg, _BF16)
    b = jnp.concatenate([zc, a, zc], axis=1)
    b = jnp.maximum(jnp.maximum(b[:, 0:112], b[:, 1:113]), b[:, 2:114])
    o_ref[0] = b.reshape(56, 56, 2, 128)[:, :, 0]


def _im2col7(x):
    B = x.shape[0]
    xp = jnp.pad(x, ((0, 0), (3, 3), (3, 3), (0, 0)))
    wcols = [lax.slice(xp, (0, 0, j, 0), (B, 230, j + 223, 3), (1, 1, 2, 1))
             for j in range(7)]
    xw = jnp.stack(wcols, axis=3).reshape(B, 230, 112, 21)
    hcols = [lax.slice(xw, (0, i, 0, 0), (B, i + 223, 112, 21), (1, 2, 1, 1))
             for i in range(7)]
    return jnp.stack(hcols, axis=3).reshape(B, 112 * 112, 147)


def _conv1_pool(x_nhwc, w, b):
    B = x_nhwc.shape[0]
    patches = _im2col7(x_nhwc)
    w = w[:147]
    return pl.pallas_call(
        _conv1_pool_kernel,
        out_shape=jax.ShapeDtypeStruct((B, 56, 56, 128), _BF16),
        grid=(B,),
        in_specs=[pl.BlockSpec((1, 112 * 112, 147), lambda i: (i, 0, 0)),
                  pl.BlockSpec((147, 128), lambda i: (0, 0)),
                  pl.BlockSpec((1, 128), lambda i: (0, 0))],
        out_specs=pl.BlockSpec((1, 56, 56, 128), lambda i: (i, 0, 0, 0)),
        compiler_params=pltpu.CompilerParams(
            dimension_semantics=("parallel",),
            vmem_limit_bytes=48 * _MB),
    )(patches, w, b)


def _head_kernel(x_ref, w_ref, b_ref, l_ref, loss_ref, pred_ref):
    pooled = jnp.mean(x_ref[...].astype(jnp.float32), axis=1)
    logits = jnp.dot(pooled.astype(_BF16), w_ref[...],
                     preferred_element_type=jnp.float32) + b_ref[...]
    B, Np = logits.shape
    col = lax.broadcasted_iota(jnp.int32, (B, Np), 1)
    lg = jnp.where(col < 1000, logits, -jnp.inf)
    row_max = jnp.max(lg, axis=-1, keepdims=True)
    lse = jnp.log(jnp.sum(jnp.exp(lg - row_max), axis=-1,
                          keepdims=True)) + row_max
    onehot = (col == l_ref[...]).astype(jnp.float32)
    picked = jnp.sum(logits * onehot, axis=-1, keepdims=True)
    loss_ref[...] = jnp.sum(lse - picked, axis=0, keepdims=True) / B
    masked = jnp.where(lg == row_max, col, Np)
    pred_ref[...] = jnp.min(masked, axis=-1, keepdims=True)


def _head(x, fc_w, fc_b, labels):
    B, HW, C = x.shape
    Np = fc_w.shape[1]
    labels2 = labels.reshape(B, 1).astype(jnp.int32)
    loss, pred = pl.pallas_call(
        _head_kernel,
        out_shape=(jax.ShapeDtypeStruct((1, 1), jnp.float32),
                   jax.ShapeDtypeStruct((B, 1), jnp.int32)),
        grid=(1,),
        in_specs=[pl.BlockSpec((B, HW, C), lambda i: (0, 0, 0)),
                  pl.BlockSpec((C, Np), lambda i: (0, 0)),
                  pl.BlockSpec((1, Np), lambda i: (0, 0)),
                  pl.BlockSpec((B, 1), lambda i: (0, 0))],
        out_specs=(pl.BlockSpec((1, 1), lambda i: (0, 0)),
                   pl.BlockSpec((B, 1), lambda i: (0, 0))),
    )(x, fc_w, fc_b, labels2)
    return loss[0, 0], pred


def kernel(conv1_w, conv1_b,
           w1_0_0, b1_0_0, w2_0_0, b2_0_0,
           w1_0_1, b1_0_1, w2_0_1, b2_0_1,
           w1_1_0, b1_1_0, mask1_1_0, w2_1_0, b2_1_0,
           w1_1_1, b1_1_1, w2_1_1, b2_1_1,
           w1_2_0, b1_2_0, mask1_2_0, w2_2_0, b2_2_0,
           w1_2_1, b1_2_1, w2_2_1, b2_2_1,
           w1_3_0, b1_3_0, mask1_3_0, w2_3_0, b2_3_0,
           w1_3_1, b1_3_1, w2_3_1, b2_3_1,
           fc_w, fc_b, x, labels):
    del mask1_1_0, mask1_2_0, mask1_3_0
    x_nhwc = jnp.transpose(x.astype(_BF16), (0, 2, 3, 1))

    h = _conv1_pool(x_nhwc, conv1_w, conv1_b)

    h = _run_block_a(h, w1_0_0, b1_0_0, w2_0_0, b2_0_0, tb=2, vmem_mb=48)
    h = _run_block_a(h, w1_0_1, b1_0_1, w2_0_1, b2_0_1, tb=2, vmem_mb=48)

    h = _run_block_b(h, w1_1_0, b1_1_0, w2_1_0, b2_1_0, cout=128, tb=4,
                     vmem_mb=48)
    h = _run_block_a(h, w1_1_1, b1_1_1, w2_1_1, b2_1_1, tb=8, vmem_mb=48)

    h = _run_block_b(h, w1_2_0, b1_2_0, w2_2_0, b2_2_0, cout=256, tb=8,
                     vmem_mb=48)
    h = _run_block_a(h, w1_2_1, b1_2_1, w2_2_1, b2_2_1, tb=8, vmem_mb=48)

    h = _run_block_b(h, w1_3_0, b1_3_0, w2_3_0, b2_3_0, cout=512, tb=16,
                     vmem_mb=48)
    h = _run_block_a(h, w1_3_1, b1_3_1, w2_3_1, b2_3_1, tb=16, vmem_mb=48)

    B = h.shape[0]
    loss, preds = _head(h.reshape(B, 49, 512), fc_w, fc_b, labels)
    out = {'y_pred': preds, 'y_true': labels.reshape(-1, 1)}
    return out, loss
